# baseline (speedup 1.0000x reference)
import functools

import jax
import jax.numpy as jnp
from jax import lax
from jax.experimental import pallas as pl
from jax.experimental.pallas import tpu as pltpu
from jax.experimental.pallas import tpu_sc as plsc

_EMBED = 1024
_E = 16
_B = 128

_NW = 32
_ROWS = _E * _EMBED
_RPW = _ROWS // _NW
_CHUNK = 32
_NCHUNK = _RPW // _CHUNK
_LANES = 16


def _rowsum_sc(W_flat):
    mesh = plsc.VectorSubcoreMesh(core_axis_name="c", subcore_axis_name="s")

    @functools.partial(
        pl.kernel,
        mesh=mesh,
        out_type=jax.ShapeDtypeStruct((_ROWS,), jnp.float32),
        scratch_types=[
            pltpu.VMEM((_CHUNK * _EMBED,), jnp.float32),
            pltpu.VMEM((_CHUNK * _EMBED,), jnp.float32),
            pltpu.VMEM((_RPW,), jnp.float32),
            pltpu.SemaphoreType.DMA,
            pltpu.SemaphoreType.DMA,
        ],
        compiler_params=pltpu.CompilerParams(needs_layout_passes=False),
    )
    def k(w_hbm, out_hbm, buf0, buf1, res, sem0, sem1):
        wid = lax.axis_index("s") * 2 + lax.axis_index("c")
        base = wid * _RPW * _EMBED
        bufs = (buf0, buf1)
        sems = (sem0, sem1)

        def start(c):
            off = base + c * _CHUNK * _EMBED
            return pltpu.async_copy(
                w_hbm.at[pl.ds(off, _CHUNK * _EMBED)], bufs[c % 2], sems[c % 2])

        cp = start(0)
        for c in range(_NCHUNK):
            nxt = start(c + 1) if c + 1 < _NCHUNK else None
            cp.wait()
            buf = bufs[c % 2]
            for grp in range(_CHUNK // _LANES):
                row_base = (grp * _LANES
                            + lax.iota(jnp.int32, _LANES)) * _EMBED

                def body(j, carry):
                    acc, idx = carry
                    for _ in range(16):
                        acc = acc + plsc.load_gather(buf, [idx])
                        idx = idx + 1
                    return acc, idx

                acc0 = jnp.zeros((_LANES,), jnp.float32)
                acc, _ = lax.fori_loop(0, _EMBED // 16, body, (acc0, row_base))
                res[pl.ds(c * _CHUNK + grp * _LANES, _LANES)] = acc
            cp = nxt
        pltpu.sync_copy(res, out_hbm.at[pl.ds(wid * _RPW, _RPW)])

    return k(W_flat)


def _combine_kernel(x_ref, Wg_ref, bg_ref, ws_ref, b_ref, out_ref):
    logits = x_ref[...] @ Wg_ref[...] + bg_ref[...]
    m = jnp.max(logits, axis=1, keepdims=True)
    p = jnp.exp(logits - m)
    g = 1.0 / jnp.sum(p, axis=1)
    ii = jax.lax.broadcasted_iota(jnp.int32, (_B, _E), 1)
    idx = jnp.min(jnp.where(logits == m, ii, _E), axis=1)
    S = lax.dot_general(x_ref[...], ws_ref[...],
                        (((1,), (1,)), ((), ())))
    bsum = jnp.sum(b_ref[...], axis=1)
    A = g[:, None] * (S + bsum[None, :])
    H = (ii == idx[:, None]).astype(jnp.float32)
    out_ref[...] = A @ H.T


def kernel(x, Wg, bg, W, b):
    w_sum = _rowsum_sc(W.reshape(_ROWS * _EMBED)).reshape(_E, _EMBED)
    return pl.pallas_call(
        _combine_kernel,
        out_shape=jax.ShapeDtypeStruct((_B, _B), jnp.float32),
    )(x, Wg, bg.reshape(1, _E), w_sum, b)


# speedup vs baseline: 2.5417x; 2.5417x over previous
import functools

import jax
import jax.numpy as jnp
from jax import lax
from jax.experimental import pallas as pl
from jax.experimental.pallas import tpu as pltpu
from jax.experimental.pallas import tpu_sc as plsc

_EMBED = 1024
_E = 16
_B = 128

_NW = 32
_ROWS = _E * _EMBED
_RPW = _ROWS // _NW
_CHUNK = 32
_NCHUNK = _RPW // _CHUNK
_LANES = 16


def _rowsum_sc(W_flat):
    mesh = plsc.VectorSubcoreMesh(core_axis_name="c", subcore_axis_name="s")

    @functools.partial(
        pl.kernel,
        mesh=mesh,
        out_type=jax.ShapeDtypeStruct((_ROWS,), jnp.float32),
        scratch_types=[
            pltpu.VMEM((_CHUNK * _EMBED,), jnp.float32),
            pltpu.VMEM((_CHUNK * _EMBED,), jnp.float32),
            pltpu.VMEM((_RPW,), jnp.float32),
            pltpu.SemaphoreType.DMA,
            pltpu.SemaphoreType.DMA,
        ],
        compiler_params=pltpu.CompilerParams(needs_layout_passes=False),
    )
    def k(w_hbm, out_hbm, buf0, buf1, res, sem0, sem1):
        wid = lax.axis_index("s") * 2 + lax.axis_index("c")
        base = wid * _RPW * _EMBED
        bufs = (buf0, buf1)
        sems = (sem0, sem1)

        def start(c):
            off = base + c * _CHUNK * _EMBED
            return pltpu.async_copy(
                w_hbm.at[pl.ds(off, _CHUNK * _EMBED)], bufs[c % 2], sems[c % 2])

        lane = lax.iota(jnp.int32, _LANES)

        def lanesum(v):
            for d in (8, 4, 2, 1):
                v = v + v.at[lane ^ d].get(mode="promise_in_bounds",
                                           unique_indices=True)
            return v

        cp = start(0)
        for c in range(_NCHUNK):
            nxt = start(c + 1) if c + 1 < _NCHUNK else None
            cp.wait()
            buf = bufs[c % 2]

            def row_body(r, res_vec, buf=buf, c=c):
                off = r * _EMBED
                accs = [jnp.zeros((_LANES,), jnp.float32) for _ in range(4)]
                for k in range(_EMBED // _LANES):
                    accs[k % 4] = accs[k % 4] + buf[pl.ds(off + k * _LANES,
                                                          _LANES)]
                tot = lanesum((accs[0] + accs[1]) + (accs[2] + accs[3]))
                res_vec = jnp.where(lane == (r % _LANES), tot, res_vec)

                @pl.when(r % _LANES == _LANES - 1)
                def _():
                    g16 = (r // _LANES) * _LANES
                    res[pl.ds(c * _CHUNK + g16, _LANES)] = res_vec
                return res_vec

            lax.fori_loop(0, _CHUNK, row_body,
                          jnp.zeros((_LANES,), jnp.float32))
            cp = nxt
        pltpu.sync_copy(res, out_hbm.at[pl.ds(wid * _RPW, _RPW)])

    return k(W_flat)


def _combine_kernel(x_ref, Wg_ref, bg_ref, ws_ref, b_ref, out_ref):
    logits = x_ref[...] @ Wg_ref[...] + bg_ref[...]
    m = jnp.max(logits, axis=1, keepdims=True)
    p = jnp.exp(logits - m)
    g = 1.0 / jnp.sum(p, axis=1)
    ii = jax.lax.broadcasted_iota(jnp.int32, (_B, _E), 1)
    idx = jnp.min(jnp.where(logits == m, ii, _E), axis=1)
    S = lax.dot_general(x_ref[...], ws_ref[...],
                        (((1,), (1,)), ((), ())))
    bsum = jnp.sum(b_ref[...], axis=1)
    A = g[:, None] * (S + bsum[None, :])
    H = (ii == idx[:, None]).astype(jnp.float32)
    out_ref[...] = A @ H.T


def kernel(x, Wg, bg, W, b):
    w_sum = _rowsum_sc(W.reshape(_ROWS * _EMBED)).reshape(_E, _EMBED)
    return pl.pallas_call(
        _combine_kernel,
        out_shape=jax.ShapeDtypeStruct((_B, _B), jnp.float32),
    )(x, Wg, bg.reshape(1, _E), w_sum, b)
